# baseline (speedup 1.0000x reference)
import functools

import jax
import jax.numpy as jnp
from jax import lax
from jax.experimental import pallas as pl
from jax.experimental.pallas import tpu as pltpu
from jax.experimental.pallas import tpu_sc as plsc

N = 10000
NUM_CLASSES = 40
NPAD = 10240
NC, NS, LANES = 2, 16, 16
NW = NC * NS
CHUNK = 128
E = 320000
NCHUNK = 80
EPT = NCHUNK * CHUNK
EPAD = NW * EPT
RPT = NPAD // NS
ROWBLK = 1024
GRID = NPAD // ROWBLK
DUMMY = NPAD - 1
H1P = 104
H2P = 48
H3P = 48
DEGW = 16


def _sc_mesh():
    return plsc.VectorSubcoreMesh(core_axis_name="c", subcore_axis_name="s")


def _sc_degree(col2):
    @functools.partial(
        pl.kernel,
        out_type=jax.ShapeDtypeStruct((NC, NPAD, DEGW), jnp.float32),
        mesh=_sc_mesh(),
        scratch_types=[
            pltpu.VMEM((NCHUNK, CHUNK), jnp.int32),
            pltpu.VMEM((CHUNK, DEGW), jnp.float32),
            pltpu.VMEM((CHUNK, DEGW), jnp.float32),
            pltpu.VMEM_SHARED((NPAD, DEGW), jnp.float32),
        ],
        compiler_params=pltpu.CompilerParams(use_tc_tiling_on_sc=False),
    )
    def kern(col_hbm, out_hbm, col_v, ones_v, zbuf, acc_sh):
        c = lax.axis_index("c")
        s = lax.axis_index("s")
        wid = c * NS + s

        def fill(i, _):
            ones_v[i, :] = jnp.ones((DEGW,), jnp.float32)
            zbuf[i, :] = jnp.zeros((DEGW,), jnp.float32)
            return 0

        lax.fori_loop(0, CHUNK, fill, 0)

        def zloop(i, _):
            pltpu.sync_copy(zbuf, acc_sh.at[pl.ds(s * RPT + i * CHUNK, CHUNK)])
            return 0

        lax.fori_loop(0, RPT // CHUNK, zloop, 0)
        pltpu.sync_copy(col_hbm.at[pl.ds(wid * NCHUNK, NCHUNK)], col_v)
        plsc.subcore_barrier()

        def body(j, _):
            pltpu.sync_copy(ones_v, acc_sh.at[col_v.at[j]], add=True)
            return 0

        lax.fori_loop(0, NCHUNK, body, 0)
        plsc.subcore_barrier()

        def wloop(i, _):
            r0 = s * RPT + i * CHUNK
            pltpu.sync_copy(acc_sh.at[pl.ds(r0, CHUNK)], zbuf)
            pltpu.sync_copy(zbuf, out_hbm.at[c].at[pl.ds(r0, CHUNK)])
            return 0

        lax.fori_loop(0, RPT // CHUNK, wloop, 0)

    return kern(col2)


def _sc_gather_scatter(h, row2, col2, hp, nbuf, nch0, nch1):
    assert nch0 % nbuf == 0 and nch1 % nbuf == 0
    assert 16 * (nch0 + nch1) * CHUNK == EPAD
    nchmax = max(nch0, nch1)

    @functools.partial(
        pl.kernel,
        out_type=jax.ShapeDtypeStruct((NC, NPAD, hp), jnp.float32),
        mesh=_sc_mesh(),
        scratch_types=[
            pltpu.VMEM((nchmax, CHUNK), jnp.int32),
            pltpu.VMEM((nchmax, CHUNK), jnp.int32),
            pltpu.VMEM((nbuf, CHUNK, hp), jnp.float32),
            pltpu.VMEM_SHARED((NPAD, hp), jnp.float32),
            pltpu.SemaphoreType.DMA((nbuf,)),
            pltpu.SemaphoreType.DMA((nbuf,)),
        ],
        compiler_params=pltpu.CompilerParams(use_tc_tiling_on_sc=False),
    )
    def kern(h_hbm, row_hbm, col_hbm, out_hbm, row_v, col_v, buf, acc_sh,
             gsem, ssem):
        c = lax.axis_index("c")
        s = lax.axis_index("s")

        def fill(i, _):
            for k in range(-(-hp // LANES)):
                start = min(k * LANES, hp - LANES)
                buf[0, i, pl.ds(start, LANES)] = jnp.zeros(
                    (LANES,), jnp.float32)
            return 0

        lax.fori_loop(0, CHUNK, fill, 0)

        def zloop(i, _):
            pltpu.sync_copy(
                buf.at[0], acc_sh.at[pl.ds(s * RPT + i * CHUNK, CHUNK)])
            return 0

        lax.fori_loop(0, RPT // CHUNK, zloop, 0)

        def run_pipe(nch, off):
            pltpu.sync_copy(row_hbm.at[pl.ds(off, nch)],
                            row_v.at[pl.ds(0, nch)])
            pltpu.sync_copy(col_hbm.at[pl.ds(off, nch)],
                            col_v.at[pl.ds(0, nch)])
            for b in range(nbuf):
                pltpu.async_copy(h_hbm.at[row_v.at[b]], buf.at[b],
                                 gsem.at[b])

            def body(t, _):
                for b in range(nbuf):
                    jj = nbuf * t + b
                    nb = (b + 1) % nbuf
                    pltpu.make_async_copy(
                        h_hbm.at[row_v.at[jj]], buf.at[b],
                        gsem.at[b]).wait()
                    pltpu.async_copy(
                        buf.at[b], acc_sh.at[col_v.at[jj]], ssem.at[b],
                        add=True)
                    nxt = jj + 1

                    @pl.when((jj >= nbuf - 1) & (nxt < nch))
                    def _():
                        pltpu.make_async_copy(
                            buf.at[nb], acc_sh.at[col_v.at[0]],
                            ssem.at[nb]).wait()
                        pltpu.async_copy(
                            h_hbm.at[row_v.at[nxt]], buf.at[nb],
                            gsem.at[nb])
                return 0

            lax.fori_loop(0, nch // nbuf, body, 0)
            for b in range(nbuf):
                pltpu.make_async_copy(
                    buf.at[b], acc_sh.at[col_v.at[0]], ssem.at[b]).wait()

        plsc.subcore_barrier()
        pl.when(c == 0)(lambda: run_pipe(nch0, s * nch0))
        if nch1 > 0:
            pl.when(c == 1)(lambda: run_pipe(nch1, NS * nch0 + s * nch1))
        plsc.subcore_barrier()

        def wloop(i, _):
            r0 = s * RPT + i * CHUNK
            pltpu.sync_copy(acc_sh.at[pl.ds(r0, CHUNK)], buf.at[0])
            pltpu.sync_copy(buf.at[0], out_hbm.at[c].at[pl.ds(r0, CHUNK)])
            return 0

        lax.fori_loop(0, RPT // CHUNK, wloop, 0)

    return kern(h, row2, col2)


def _tc1(degp, xp, w1p):
    def body(degp_ref, x_ref, w_ref, h_ref, d_ref):
        dsum = degp_ref[0] + degp_ref[1] + 1.0
        d = lax.rsqrt(dsum)
        d_ref[...] = d
        h = jnp.dot(x_ref[...], w_ref[...],
                    preferred_element_type=jnp.float32)
        h_ref[...] = h * d[:, 0:1]

    return pl.pallas_call(
        body,
        grid=(GRID,),
        in_specs=[
            pl.BlockSpec((2, ROWBLK, DEGW), lambda i: (0, i, 0)),
            pl.BlockSpec((ROWBLK, 128), lambda i: (i, 0)),
            pl.BlockSpec((128, H1P), lambda i: (0, 0)),
        ],
        out_specs=[
            pl.BlockSpec((ROWBLK, H1P), lambda i: (i, 0)),
            pl.BlockSpec((ROWBLK, DEGW), lambda i: (i, 0)),
        ],
        out_shape=[
            jax.ShapeDtypeStruct((NPAD, H1P), jnp.float32),
            jax.ShapeDtypeStruct((NPAD, DEGW), jnp.float32),
        ],
    )(degp, xp, w1p)


def _tc2(acc1, h1p, dd, topop, w2ap, w2bp, b1p):
    def body(a_ref, h_ref, d_ref, t_ref, wa_ref, wb_ref, b_ref, o_ref):
        d = d_ref[...][:, 0:1]
        pre = (a_ref[0] + a_ref[1] + h_ref[...]) * d + b_ref[...]
        o1 = jnp.maximum(pre, 0.0)
        h2 = (jnp.dot(o1, wa_ref[...], preferred_element_type=jnp.float32)
              + jnp.dot(t_ref[...], wb_ref[...],
                        preferred_element_type=jnp.float32))
        o_ref[...] = h2 * d

    return pl.pallas_call(
        body,
        grid=(GRID,),
        in_specs=[
            pl.BlockSpec((2, ROWBLK, H1P), lambda i: (0, i, 0)),
            pl.BlockSpec((ROWBLK, H1P), lambda i: (i, 0)),
            pl.BlockSpec((ROWBLK, DEGW), lambda i: (i, 0)),
            pl.BlockSpec((ROWBLK, 16), lambda i: (i, 0)),
            pl.BlockSpec((H1P, H2P), lambda i: (0, 0)),
            pl.BlockSpec((16, H2P), lambda i: (0, 0)),
            pl.BlockSpec((1, H1P), lambda i: (0, 0)),
        ],
        out_specs=pl.BlockSpec((ROWBLK, H2P), lambda i: (i, 0)),
        out_shape=jax.ShapeDtypeStruct((NPAD, H2P), jnp.float32),
    )(acc1, h1p, dd, topop, w2ap, w2bp, b1p)


def _tc3(acc2, h2p, dd, w3p, b2p):
    def body(a_ref, h_ref, d_ref, w_ref, b_ref, o_ref):
        d = d_ref[...][:, 0:1]
        pre = (a_ref[0] + a_ref[1] + h_ref[...]) * d + b_ref[...]
        o2 = jnp.maximum(pre, 0.0)
        h3 = jnp.dot(o2, w_ref[...], preferred_element_type=jnp.float32)
        o_ref[...] = h3 * d

    return pl.pallas_call(
        body,
        grid=(GRID,),
        in_specs=[
            pl.BlockSpec((2, ROWBLK, H2P), lambda i: (0, i, 0)),
            pl.BlockSpec((ROWBLK, H2P), lambda i: (i, 0)),
            pl.BlockSpec((ROWBLK, DEGW), lambda i: (i, 0)),
            pl.BlockSpec((H2P, H3P), lambda i: (0, 0)),
            pl.BlockSpec((1, H2P), lambda i: (0, 0)),
        ],
        out_specs=pl.BlockSpec((ROWBLK, H3P), lambda i: (i, 0)),
        out_shape=jax.ShapeDtypeStruct((NPAD, H3P), jnp.float32),
    )(acc2, h2p, dd, w3p, b2p)


def _tc4(acc3, h3p, dd, b3p):
    def body(a_ref, h_ref, d_ref, b_ref, o_ref):
        d = d_ref[...][:, 0:1]
        logits = (a_ref[0] + a_ref[1] + h_ref[...]) * d + b_ref[...]
        lg = logits[:, :NUM_CLASSES]
        m = jnp.max(lg, axis=1, keepdims=True)
        lse = jnp.log(jnp.sum(jnp.exp(lg - m), axis=1, keepdims=True)) + m
        o_ref[...] = lg - lse

    return pl.pallas_call(
        body,
        grid=(GRID,),
        in_specs=[
            pl.BlockSpec((2, ROWBLK, H3P), lambda i: (0, i, 0)),
            pl.BlockSpec((ROWBLK, H3P), lambda i: (i, 0)),
            pl.BlockSpec((ROWBLK, DEGW), lambda i: (i, 0)),
            pl.BlockSpec((1, H3P), lambda i: (0, 0)),
        ],
        out_specs=pl.BlockSpec((ROWBLK, NUM_CLASSES), lambda i: (i, 0)),
        out_shape=jax.ShapeDtypeStruct((NPAD, NUM_CLASSES), jnp.float32),
    )(acc3, h3p, dd, b3p)


def kernel(x, edge_index, topo_mx, W1, b1, W2, b2, W3, b3):
    row = edge_index[0].astype(jnp.int32)
    col = edge_index[1].astype(jnp.int32)
    pad = EPAD - E
    row2 = jnp.concatenate(
        [row, jnp.zeros((pad,), jnp.int32)]).reshape(EPAD // CHUNK, CHUNK)
    pad_dst = N + jnp.arange(pad, dtype=jnp.int32) % (NPAD - N)
    col2 = jnp.concatenate([col, pad_dst]).reshape(EPAD // CHUNK, CHUNK)
    xp = jnp.pad(x, ((0, NPAD - N), (0, 0)))
    topop = jnp.pad(topo_mx, ((0, NPAD - N), (0, 0)))
    h1 = W1.shape[1]
    w1p = jnp.pad(W1, ((0, 0), (0, H1P - h1)))
    b1p = jnp.pad(b1, (0, H1P - h1)).reshape(1, H1P)
    h2 = W2.shape[1]
    w2ap = jnp.pad(W2[:h1], ((0, H1P - h1), (0, H2P - h2)))
    w2bp = jnp.pad(W2[h1:], ((0, 0), (0, H2P - h2)))
    b2p = jnp.pad(b2, (0, H2P - h2)).reshape(1, H2P)
    w3p = jnp.pad(W3, ((0, H2P - h2), (0, H3P - NUM_CLASSES)))
    b3p = jnp.pad(b3, (0, H3P - NUM_CLASSES)).reshape(1, H3P)

    degp = _sc_degree(col2)
    h1p_, dd = _tc1(degp, xp, w1p)
    acc1 = _sc_gather_scatter(h1p_, row2, col2, H1P, 2, 144, 16)
    h2p_ = _tc2(acc1, h1p_, dd, topop, w2ap, w2bp, b1p)
    acc2 = _sc_gather_scatter(h2p_, row2, col2, H2P, 8, 160, 0)
    h3p_ = _tc3(acc2, h2p_, dd, w3p, b2p)
    acc3 = _sc_gather_scatter(h3p_, row2, col2, H3P, 8, 160, 0)
    out = _tc4(acc3, h3p_, dd, b3p)
    return out[:N]


# speedup vs baseline: 1.2657x; 1.2657x over previous
import functools

import jax
import jax.numpy as jnp
from jax import lax
from jax.experimental import pallas as pl
from jax.experimental.pallas import tpu as pltpu
from jax.experimental.pallas import tpu_sc as plsc

N = 10000
NUM_CLASSES = 40
NPAD = 10240
NC, NS, LANES = 2, 16, 16
NW = NC * NS
CHUNK = 128
E = 320000
NCHUNK = 80
EPT = NCHUNK * CHUNK
EPAD = NW * EPT
RPT = NPAD // NS
ROWBLK = 1024
GRID = NPAD // ROWBLK
DUMMY = NPAD - 1
H1P = 104
H2P = 48
H3P = 48
DEGW = 16


def _sc_mesh():
    return plsc.VectorSubcoreMesh(core_axis_name="c", subcore_axis_name="s")


def _sc_degree(col2):
    @functools.partial(
        pl.kernel,
        out_type=jax.ShapeDtypeStruct((NC, NPAD, DEGW), jnp.float32),
        mesh=_sc_mesh(),
        scratch_types=[
            pltpu.VMEM((NCHUNK, CHUNK), jnp.int32),
            pltpu.VMEM((CHUNK, DEGW), jnp.float32),
            pltpu.VMEM((CHUNK, DEGW), jnp.float32),
            pltpu.VMEM_SHARED((NPAD, DEGW), jnp.float32),
        ],
        compiler_params=pltpu.CompilerParams(use_tc_tiling_on_sc=False),
    )
    def kern(col_hbm, out_hbm, col_v, ones_v, zbuf, acc_sh):
        c = lax.axis_index("c")
        s = lax.axis_index("s")
        wid = c * NS + s

        def fill(i, _):
            ones_v[i, :] = jnp.ones((DEGW,), jnp.float32)
            zbuf[i, :] = jnp.zeros((DEGW,), jnp.float32)
            return 0

        lax.fori_loop(0, CHUNK, fill, 0)

        def zloop(i, _):
            pltpu.sync_copy(zbuf, acc_sh.at[pl.ds(s * RPT + i * CHUNK, CHUNK)])
            return 0

        lax.fori_loop(0, RPT // CHUNK, zloop, 0)
        pltpu.sync_copy(col_hbm.at[pl.ds(wid * NCHUNK, NCHUNK)], col_v)
        plsc.subcore_barrier()

        def body(j, _):
            pltpu.sync_copy(ones_v, acc_sh.at[col_v.at[j]], add=True)
            return 0

        lax.fori_loop(0, NCHUNK, body, 0)
        plsc.subcore_barrier()

        def wloop(i, _):
            r0 = s * RPT + i * CHUNK
            pltpu.sync_copy(acc_sh.at[pl.ds(r0, CHUNK)], zbuf)
            pltpu.sync_copy(zbuf, out_hbm.at[c].at[pl.ds(r0, CHUNK)])
            return 0

        lax.fori_loop(0, RPT // CHUNK, wloop, 0)

    return kern(col2)


def _sc_gather_scatter(h, row2, col2, hp, nbuf, nch0, nch1):
    assert nch0 % nbuf == 0 and nch1 % nbuf == 0
    assert 16 * (nch0 + nch1) * CHUNK == EPAD
    nchmax = max(nch0, nch1)

    @functools.partial(
        pl.kernel,
        out_type=jax.ShapeDtypeStruct((NC, NPAD, hp), jnp.float32),
        mesh=_sc_mesh(),
        scratch_types=[
            pltpu.VMEM((nchmax, CHUNK), jnp.int32),
            pltpu.VMEM((nchmax, CHUNK), jnp.int32),
            pltpu.VMEM((nbuf, CHUNK, hp), jnp.float32),
            pltpu.VMEM_SHARED((NPAD, hp), jnp.float32),
            pltpu.SemaphoreType.DMA((nbuf,)),
            pltpu.SemaphoreType.DMA((nbuf,)),
        ],
        compiler_params=pltpu.CompilerParams(use_tc_tiling_on_sc=False),
    )
    def kern(h_hbm, row_hbm, col_hbm, out_hbm, row_v, col_v, buf, acc_sh,
             gsem, ssem):
        c = lax.axis_index("c")
        s = lax.axis_index("s")

        def fill(i, _):
            for k in range(-(-hp // LANES)):
                start = min(k * LANES, hp - LANES)
                buf[0, i, pl.ds(start, LANES)] = jnp.zeros(
                    (LANES,), jnp.float32)
            return 0

        lax.fori_loop(0, CHUNK, fill, 0)

        def zloop(i, _):
            pltpu.sync_copy(
                buf.at[0], acc_sh.at[pl.ds(s * RPT + i * CHUNK, CHUNK)])
            return 0

        lax.fori_loop(0, RPT // CHUNK, zloop, 0)

        def run_pipe(nch, off):
            pltpu.sync_copy(row_hbm.at[pl.ds(off, nch)],
                            row_v.at[pl.ds(0, nch)])
            pltpu.sync_copy(col_hbm.at[pl.ds(off, nch)],
                            col_v.at[pl.ds(0, nch)])
            for b in range(nbuf):
                pltpu.async_copy(h_hbm.at[row_v.at[b]], buf.at[b],
                                 gsem.at[b])

            def body(t, _):
                for b in range(nbuf):
                    jj = nbuf * t + b
                    nb = (b + 1) % nbuf
                    pltpu.make_async_copy(
                        h_hbm.at[row_v.at[jj]], buf.at[b],
                        gsem.at[b]).wait()
                    pltpu.async_copy(
                        buf.at[b], acc_sh.at[col_v.at[jj]], ssem.at[b],
                        add=True)
                    nxt = jj + 1

                    @pl.when((jj >= nbuf - 1) & (nxt < nch))
                    def _():
                        pltpu.make_async_copy(
                            buf.at[nb], acc_sh.at[col_v.at[0]],
                            ssem.at[nb]).wait()
                        pltpu.async_copy(
                            h_hbm.at[row_v.at[nxt]], buf.at[nb],
                            gsem.at[nb])
                return 0

            lax.fori_loop(0, nch // nbuf, body, 0)
            for b in range(nbuf):
                pltpu.make_async_copy(
                    buf.at[b], acc_sh.at[col_v.at[0]], ssem.at[b]).wait()

        plsc.subcore_barrier()
        pl.when(c == 0)(lambda: run_pipe(nch0, s * nch0))
        if nch1 > 0:
            pl.when(c == 1)(lambda: run_pipe(nch1, NS * nch0 + s * nch1))
        plsc.subcore_barrier()

        def wloop(i, _):
            r0 = s * RPT + i * CHUNK
            pltpu.sync_copy(acc_sh.at[pl.ds(r0, CHUNK)], buf.at[0])
            pltpu.sync_copy(buf.at[0], out_hbm.at[c].at[pl.ds(r0, CHUNK)])
            return 0

        lax.fori_loop(0, RPT // CHUNK, wloop, 0)

    return kern(h, row2, col2)


def _tc1(degp, xp, w1p):
    def body(degp_ref, x_ref, w_ref, h_ref, d_ref):
        dsum = degp_ref[0] + degp_ref[1] + 1.0
        d = lax.rsqrt(dsum)
        d_ref[...] = d
        h = jnp.dot(x_ref[...], w_ref[...],
                    preferred_element_type=jnp.float32)
        h_ref[...] = h * d[:, 0:1]

    return pl.pallas_call(
        body,
        grid=(GRID,),
        in_specs=[
            pl.BlockSpec((2, ROWBLK, DEGW), lambda i: (0, i, 0)),
            pl.BlockSpec((ROWBLK, 128), lambda i: (i, 0)),
            pl.BlockSpec((128, H1P), lambda i: (0, 0)),
        ],
        out_specs=[
            pl.BlockSpec((ROWBLK, H1P), lambda i: (i, 0)),
            pl.BlockSpec((ROWBLK, DEGW), lambda i: (i, 0)),
        ],
        out_shape=[
            jax.ShapeDtypeStruct((NPAD, H1P), jnp.float32),
            jax.ShapeDtypeStruct((NPAD, DEGW), jnp.float32),
        ],
    )(degp, xp, w1p)


def _tc2(acc1, h1p, dd, topop, w2ap, w2bp, b1p):
    def body(a_ref, h_ref, d_ref, t_ref, wa_ref, wb_ref, b_ref, o_ref):
        d = d_ref[...][:, 0:1]
        pre = (a_ref[0] + a_ref[1] + h_ref[...]) * d + b_ref[...]
        o1 = jnp.maximum(pre, 0.0)
        h2 = (jnp.dot(o1, wa_ref[...], preferred_element_type=jnp.float32)
              + jnp.dot(t_ref[...], wb_ref[...],
                        preferred_element_type=jnp.float32))
        o_ref[...] = h2 * d

    return pl.pallas_call(
        body,
        grid=(GRID,),
        in_specs=[
            pl.BlockSpec((2, ROWBLK, H1P), lambda i: (0, i, 0)),
            pl.BlockSpec((ROWBLK, H1P), lambda i: (i, 0)),
            pl.BlockSpec((ROWBLK, DEGW), lambda i: (i, 0)),
            pl.BlockSpec((ROWBLK, 16), lambda i: (i, 0)),
            pl.BlockSpec((H1P, H2P), lambda i: (0, 0)),
            pl.BlockSpec((16, H2P), lambda i: (0, 0)),
            pl.BlockSpec((1, H1P), lambda i: (0, 0)),
        ],
        out_specs=pl.BlockSpec((ROWBLK, H2P), lambda i: (i, 0)),
        out_shape=jax.ShapeDtypeStruct((NPAD, H2P), jnp.float32),
    )(acc1, h1p, dd, topop, w2ap, w2bp, b1p)


def _tc3(acc2, h2p, dd, w3p, b2p):
    def body(a_ref, h_ref, d_ref, w_ref, b_ref, o_ref):
        d = d_ref[...][:, 0:1]
        pre = (a_ref[0] + a_ref[1] + h_ref[...]) * d + b_ref[...]
        o2 = jnp.maximum(pre, 0.0)
        h3 = jnp.dot(o2, w_ref[...], preferred_element_type=jnp.float32)
        o_ref[...] = h3 * d

    return pl.pallas_call(
        body,
        grid=(GRID,),
        in_specs=[
            pl.BlockSpec((2, ROWBLK, H2P), lambda i: (0, i, 0)),
            pl.BlockSpec((ROWBLK, H2P), lambda i: (i, 0)),
            pl.BlockSpec((ROWBLK, DEGW), lambda i: (i, 0)),
            pl.BlockSpec((H2P, H3P), lambda i: (0, 0)),
            pl.BlockSpec((1, H2P), lambda i: (0, 0)),
        ],
        out_specs=pl.BlockSpec((ROWBLK, H3P), lambda i: (i, 0)),
        out_shape=jax.ShapeDtypeStruct((NPAD, H3P), jnp.float32),
    )(acc2, h2p, dd, w3p, b2p)


def _tc4(acc3, h3p, dd, b3p):
    def body(a_ref, h_ref, d_ref, b_ref, o_ref):
        d = d_ref[...][:, 0:1]
        logits = (a_ref[0] + a_ref[1] + h_ref[...]) * d + b_ref[...]
        lg = logits[:, :NUM_CLASSES]
        m = jnp.max(lg, axis=1, keepdims=True)
        lse = jnp.log(jnp.sum(jnp.exp(lg - m), axis=1, keepdims=True)) + m
        o_ref[...] = lg - lse

    return pl.pallas_call(
        body,
        grid=(GRID,),
        in_specs=[
            pl.BlockSpec((2, ROWBLK, H3P), lambda i: (0, i, 0)),
            pl.BlockSpec((ROWBLK, H3P), lambda i: (i, 0)),
            pl.BlockSpec((ROWBLK, DEGW), lambda i: (i, 0)),
            pl.BlockSpec((1, H3P), lambda i: (0, 0)),
        ],
        out_specs=pl.BlockSpec((ROWBLK, NUM_CLASSES), lambda i: (i, 0)),
        out_shape=jax.ShapeDtypeStruct((NPAD, NUM_CLASSES), jnp.float32),
    )(acc3, h3p, dd, b3p)


def kernel(x, edge_index, topo_mx, W1, b1, W2, b2, W3, b3):
    row = edge_index[0].astype(jnp.int32)
    col = edge_index[1].astype(jnp.int32)
    pad = EPAD - E
    row2 = jnp.concatenate(
        [row, jnp.zeros((pad,), jnp.int32)]).reshape(EPAD // CHUNK, CHUNK)
    pad_dst = N + jnp.arange(pad, dtype=jnp.int32) % (NPAD - N)
    col2 = jnp.concatenate([col, pad_dst]).reshape(EPAD // CHUNK, CHUNK)
    xp = jnp.pad(x, ((0, NPAD - N), (0, 0)))
    topop = jnp.pad(topo_mx, ((0, NPAD - N), (0, 0)))
    h1 = W1.shape[1]
    w1p = jnp.pad(W1, ((0, 0), (0, H1P - h1)))
    b1p = jnp.pad(b1, (0, H1P - h1)).reshape(1, H1P)
    h2 = W2.shape[1]
    w2ap = jnp.pad(W2[:h1], ((0, H1P - h1), (0, H2P - h2)))
    w2bp = jnp.pad(W2[h1:], ((0, 0), (0, H2P - h2)))
    b2p = jnp.pad(b2, (0, H2P - h2)).reshape(1, H2P)
    w3p = jnp.pad(W3, ((0, H2P - h2), (0, H3P - NUM_CLASSES)))
    b3p = jnp.pad(b3, (0, H3P - NUM_CLASSES)).reshape(1, H3P)

    degp = _sc_degree(col2)
    h1p_, dd = _tc1(degp, xp, w1p)
    acc1 = _sc_gather_scatter(h1p_, row2, col2, H1P, 2, 144, 16)
    h2p_ = _tc2(acc1, h1p_, dd, topop, w2ap, w2bp, b1p)
    acc2 = _sc_gather_scatter(h2p_, row2, col2, H2P, 8, 128, 32)
    h3p_ = _tc3(acc2, h2p_, dd, w3p, b2p)
    acc3 = _sc_gather_scatter(h3p_, row2, col2, H3P, 8, 128, 32)
    out = _tc4(acc3, h3p_, dd, b3p)
    return out[:N]
